# Initial kernel scaffold; baseline (speedup 1.0000x reference)
import jax
import jax.numpy as jnp
from jax.experimental import pallas as pl


def kernel(node_id, table):
    raise NotImplementedError("write your pallas kernel here")


# speedup vs baseline: 1.9340x; 1.9340x over previous
import functools

import jax
import jax.numpy as jnp
from jax import lax
from jax.experimental import pallas as pl
from jax.experimental.pallas import tpu as pltpu
from jax.experimental.pallas import tpu_sc as plsc

NUM_ROWS = 100000
D = 128
NC = 2
NS = 16
NW = NC * NS
C = 120
FULL = 3120
NCH = FULL // C
EXTRA_W = (NUM_ROWS - NW * FULL) // 8

_mesh = plsc.VectorSubcoreMesh(core_axis_name="c", subcore_axis_name="s")


@functools.partial(
    pl.kernel,
    mesh=_mesh,
    out_type=jax.ShapeDtypeStruct((NUM_ROWS, D), jnp.float32),
    scratch_types=[
        pltpu.VMEM((FULL + 8,), jnp.int32),
        pltpu.VMEM((C, D), jnp.float32),
        pltpu.SemaphoreType.DMA,
    ],
)
def _emb_lookup(idx_hbm, table_hbm, out_hbm, idx_v, buf, gsem):
    wid = lax.axis_index("s") * NC + lax.axis_index("c")
    base = wid * FULL + 8 * jnp.minimum(wid, EXTRA_W)

    pltpu.sync_copy(idx_hbm.at[pl.ds(base, FULL)], idx_v.at[pl.ds(0, FULL)])

    def chunk(i, carry):
        off = i * C
        pltpu.async_copy(
            table_hbm.at[idx_v.at[pl.ds(off, C)]], buf, gsem
        ).wait()
        pltpu.sync_copy(buf, out_hbm.at[pl.ds(base + off, C)])
        return carry

    lax.fori_loop(0, NCH, chunk, 0)

    @pl.when(wid < EXTRA_W)
    def _tail():
        pltpu.sync_copy(
            idx_hbm.at[pl.ds(base + FULL, 8)], idx_v.at[pl.ds(FULL, 8)]
        )
        pltpu.async_copy(
            table_hbm.at[idx_v.at[pl.ds(FULL, 8)]], buf.at[pl.ds(0, 8)], gsem
        ).wait()
        pltpu.sync_copy(buf.at[pl.ds(0, 8)], out_hbm.at[pl.ds(base + FULL, 8)])


def kernel(node_id, table):
    return _emb_lookup(node_id.astype(jnp.int32), table)


# speedup vs baseline: 2.1637x; 1.1188x over previous
import functools

import jax
import jax.numpy as jnp
from jax import lax
from jax.experimental import pallas as pl
from jax.experimental.pallas import tpu as pltpu
from jax.experimental.pallas import tpu_sc as plsc

NUM_ROWS = 100000
D = 128
NC = 2
NS = 16
NW = NC * NS
C = 120
FULL = 3120
NCH = FULL // C
EXTRA_W = (NUM_ROWS - NW * FULL) // 8

_mesh = plsc.VectorSubcoreMesh(core_axis_name="c", subcore_axis_name="s")


@functools.partial(
    pl.kernel,
    mesh=_mesh,
    out_type=jax.ShapeDtypeStruct((NUM_ROWS, D), jnp.float32),
    scratch_types=[
        pltpu.VMEM((FULL + 8,), jnp.int32),
        pltpu.VMEM((C, D), jnp.float32),
        pltpu.VMEM((C, D), jnp.float32),
        pltpu.SemaphoreType.DMA,
        pltpu.SemaphoreType.DMA,
        pltpu.SemaphoreType.DMA,
        pltpu.SemaphoreType.DMA,
    ],
)
def _emb_lookup(idx_hbm, table_hbm, out_hbm, idx_v, buf0, buf1, g0, g1, w0, w1):
    wid = lax.axis_index("s") * NC + lax.axis_index("c")
    base = wid * FULL + 8 * jnp.minimum(wid, EXTRA_W)

    pltpu.sync_copy(idx_hbm.at[pl.ds(base, FULL)], idx_v.at[pl.ds(0, FULL)])

    def gather(c, buf, sem):
        return pltpu.async_copy(table_hbm.at[idx_v.at[pl.ds(c * C, C)]], buf, sem)

    def write(c, buf, sem):
        return pltpu.async_copy(buf, out_hbm.at[pl.ds(base + c * C, C)], sem)

    gather(0, buf0, g0)

    def body(j, carry):
        c = 2 * j
        pltpu.make_async_copy(table_hbm.at[pl.ds(0, C)], buf0, g0).wait()

        @pl.when(j > 0)
        def _():
            pltpu.make_async_copy(buf1, out_hbm.at[pl.ds(0, C)], w1).wait()

        gather(c + 1, buf1, g1)
        write(c, buf0, w0)
        pltpu.make_async_copy(table_hbm.at[pl.ds(0, C)], buf1, g1).wait()

        @pl.when(j < NCH // 2 - 1)
        def _():
            pltpu.make_async_copy(buf0, out_hbm.at[pl.ds(0, C)], w0).wait()
            gather(c + 2, buf0, g0)

        write(c + 1, buf1, w1)
        return carry

    lax.fori_loop(0, NCH // 2, body, 0)
    pltpu.make_async_copy(buf0, out_hbm.at[pl.ds(0, C)], w0).wait()
    pltpu.make_async_copy(buf1, out_hbm.at[pl.ds(0, C)], w1).wait()

    @pl.when(wid < EXTRA_W)
    def _tail():
        pltpu.sync_copy(
            idx_hbm.at[pl.ds(base + FULL, 8)], idx_v.at[pl.ds(FULL, 8)]
        )
        pltpu.async_copy(
            table_hbm.at[idx_v.at[pl.ds(FULL, 8)]], buf0.at[pl.ds(0, 8)], g0
        ).wait()
        pltpu.sync_copy(buf0.at[pl.ds(0, 8)], out_hbm.at[pl.ds(base + FULL, 8)])


def kernel(node_id, table):
    return _emb_lookup(node_id.astype(jnp.int32), table)


# speedup vs baseline: 2.5293x; 1.1690x over previous
import functools

import jax
import jax.numpy as jnp
from jax import lax
from jax.experimental import pallas as pl
from jax.experimental.pallas import tpu as pltpu
from jax.experimental.pallas import tpu_sc as plsc

NUM_ROWS = 100000
D = 128
NC = 2
NS = 16
NW = NC * NS
C = 120
FULL = 3120
NCH = FULL // C
EXTRA_W = (NUM_ROWS - NW * FULL) // 8
NBUF = 4
LAG = 2

_mesh = plsc.VectorSubcoreMesh(core_axis_name="c", subcore_axis_name="s")


@functools.partial(
    pl.kernel,
    mesh=_mesh,
    out_type=jax.ShapeDtypeStruct((NUM_ROWS, D), jnp.float32),
    scratch_types=[
        pltpu.VMEM((FULL + 8,), jnp.int32),
        *[pltpu.VMEM((C, D), jnp.float32) for _ in range(NBUF)],
        *[pltpu.SemaphoreType.DMA for _ in range(2 * NBUF)],
    ],
)
def _emb_lookup(idx_hbm, table_hbm, out_hbm, idx_v, *scratch):
    bufs = scratch[:NBUF]
    gsem = scratch[NBUF:2 * NBUF]
    wsem = scratch[2 * NBUF:]
    wid = lax.axis_index("s") * NC + lax.axis_index("c")
    base = wid * FULL + 8 * jnp.minimum(wid, EXTRA_W)

    pltpu.sync_copy(idx_hbm.at[pl.ds(base, FULL)], idx_v.at[pl.ds(0, FULL)])

    for t in range(NCH + LAG):
        if t < NCH:
            b = t % NBUF
            if t >= NBUF:
                pltpu.make_async_copy(
                    bufs[b], out_hbm.at[pl.ds(0, C)], wsem[b]
                ).wait()
            pltpu.async_copy(
                table_hbm.at[idx_v.at[pl.ds(t * C, C)]], bufs[b], gsem[b]
            )
        w = t - LAG
        if 0 <= w < NCH:
            bw = w % NBUF
            pltpu.make_async_copy(
                table_hbm.at[pl.ds(0, C)], bufs[bw], gsem[bw]
            ).wait()
            pltpu.async_copy(
                bufs[bw], out_hbm.at[pl.ds(base + w * C, C)], wsem[bw]
            )
    for b in range(NBUF):
        pltpu.make_async_copy(bufs[b], out_hbm.at[pl.ds(0, C)], wsem[b]).wait()

    @pl.when(wid < EXTRA_W)
    def _tail():
        pltpu.sync_copy(
            idx_hbm.at[pl.ds(base + FULL, 8)], idx_v.at[pl.ds(FULL, 8)]
        )
        pltpu.async_copy(
            table_hbm.at[idx_v.at[pl.ds(FULL, 8)]], bufs[0].at[pl.ds(0, 8)],
            gsem[0],
        ).wait()
        pltpu.sync_copy(
            bufs[0].at[pl.ds(0, 8)], out_hbm.at[pl.ds(base + FULL, 8)]
        )


def kernel(node_id, table):
    return _emb_lookup(node_id.astype(jnp.int32), table)
